# baseline (speedup 1.0000x reference)
import functools

import jax
import jax.numpy as jnp
from jax.experimental import pallas as pl
from jax.experimental.pallas import tpu as pltpu
from jax.experimental.pallas import tpu_sc as plsc

MODEL_DIM = 2048
NUM_EXPERTS = 64
TOP_K = 8
N_TOKENS = 16384

BLOCK_TOKENS = 2048
SC_TILES = 32
SC_LANES = 16


def _router_body(x_ref, g_ref, s_ref):
    b = x_ref.shape[0]
    e = g_ref.shape[1]
    logits = jnp.dot(x_ref[...], g_ref[...], preferred_element_type=jnp.float32)
    lt = logits.T

    rowf = jax.lax.broadcasted_iota(jnp.int32, (e, b), 0).astype(jnp.float32)
    v = lt
    sel = jnp.zeros((e, b), dtype=jnp.bool_)
    for _ in range(TOP_K):
        m = jnp.max(v, axis=0, keepdims=True)
        is_m = v == m
        first = jnp.min(jnp.where(is_m, rowf, float(e)), axis=0, keepdims=True)
        pick = rowf == first
        sel = jnp.logical_or(sel, pick)
        v = jnp.where(pick, -jnp.inf, v)

    s_ref[...] = jnp.where(sel, jax.nn.sigmoid(lt), 0.0)


def _token_indices_sc(n, e):
    rows_per_tile = e // SC_TILES

    @pl.kernel(
        out_type=jax.ShapeDtypeStruct((e, n), jnp.int32),
        mesh=plsc.VectorSubcoreMesh(core_axis_name="c", subcore_axis_name="s"),
        scratch_types=[
            pltpu.VMEM((rows_per_tile, n), jnp.int32),
            pltpu.SemaphoreType.DMA,
        ],
    )
    def tok_idx(o_hbm, buf, sem):
        tile = jax.lax.axis_index("c") * 16 + jax.lax.axis_index("s")
        lane = jax.lax.iota(jnp.int32, SC_LANES)

        @pl.loop(0, n, step=SC_LANES)
        def _(c0):
            vec = c0 + lane
            for r in range(rows_per_tile):
                buf[r, pl.ds(c0, SC_LANES)] = vec

        pltpu.async_copy(
            buf, o_hbm.at[pl.ds(tile * rows_per_tile, rows_per_tile), :], sem
        ).wait()

    return tok_idx()


@functools.partial(jax.jit, static_argnames=())
def kernel(x, gate):
    n, d = x.shape
    e = gate.shape[1]
    grid = (n // BLOCK_TOKENS,)
    scores = pl.pallas_call(
        _router_body,
        grid=grid,
        in_specs=[
            pl.BlockSpec((BLOCK_TOKENS, d), lambda i: (i, 0)),
            pl.BlockSpec((d, e), lambda i: (0, 0)),
        ],
        out_specs=pl.BlockSpec((e, BLOCK_TOKENS), lambda i: (0, i)),
        out_shape=jax.ShapeDtypeStruct((e, n), jnp.float32),
    )(x, gate)
    token_idx = _token_indices_sc(n, e)
    return (scores, token_idx)


# speedup vs baseline: 1.0004x; 1.0004x over previous
import functools

import jax
import jax.numpy as jnp
from jax.experimental import pallas as pl
from jax.experimental.pallas import tpu as pltpu
from jax.experimental.pallas import tpu_sc as plsc

MODEL_DIM = 2048
NUM_EXPERTS = 64
TOP_K = 8
N_TOKENS = 16384

BLOCK_TOKENS = 2048
SC_TILES = 32
SC_LANES = 16


def _router_body(x_ref, g_ref, s_ref):
    b = x_ref.shape[0]
    e = g_ref.shape[1]
    logits = jnp.dot(x_ref[...], g_ref[...], preferred_element_type=jnp.float32)
    lt = logits.T

    rowf = jax.lax.broadcasted_iota(jnp.int32, (e, b), 0).astype(jnp.float32)
    v = lt
    sel = jnp.zeros((e, b), dtype=jnp.bool_)
    for _ in range(TOP_K):
        m = jnp.max(v, axis=0, keepdims=True)
        is_m = v == m
        first = jnp.min(jnp.where(is_m, rowf, float(e)), axis=0, keepdims=True)
        pick = rowf == first
        sel = jnp.logical_or(sel, pick)
        v = jnp.where(pick, -jnp.inf, v)

    s_ref[...] = jnp.where(sel, jax.nn.sigmoid(lt), 0.0)


def _token_indices_sc(n, e):
    rows_per_tile = e // SC_TILES

    @pl.kernel(
        out_type=jax.ShapeDtypeStruct((e, n), jnp.int32),
        mesh=plsc.VectorSubcoreMesh(core_axis_name="c", subcore_axis_name="s"),
        scratch_types=[
            pltpu.VMEM((1, n), jnp.int32),
            pltpu.SemaphoreType.DMA,
            pltpu.SemaphoreType.DMA,
        ],
    )
    def tok_idx(o_hbm, buf, sem0, sem1):
        tile = jax.lax.axis_index("c") * 16 + jax.lax.axis_index("s")
        lane = jax.lax.iota(jnp.int32, SC_LANES)

        @pl.loop(0, n, step=8 * SC_LANES)
        def _(c0):
            for j in range(8):
                off = j * SC_LANES
                buf[0, pl.ds(c0 + off, SC_LANES)] = c0 + off + lane

        r0 = tile * rows_per_tile
        copies = [
            pltpu.async_copy(buf, o_hbm.at[pl.ds(r0 + r, 1), :], sem)
            for r, sem in zip(range(rows_per_tile), (sem0, sem1))
        ]
        for cp in copies:
            cp.wait()

    return tok_idx()


@functools.partial(jax.jit, static_argnames=())
def kernel(x, gate):
    n, d = x.shape
    e = gate.shape[1]
    grid = (n // BLOCK_TOKENS,)
    scores = pl.pallas_call(
        _router_body,
        grid=grid,
        in_specs=[
            pl.BlockSpec((BLOCK_TOKENS, d), lambda i: (i, 0)),
            pl.BlockSpec((d, e), lambda i: (0, 0)),
        ],
        out_specs=pl.BlockSpec((e, BLOCK_TOKENS), lambda i: (0, i)),
        out_shape=jax.ShapeDtypeStruct((e, n), jnp.float32),
    )(x, gate)
    token_idx = _token_indices_sc(n, e)
    return (scores, token_idx)
